# baseline (speedup 1.0000x reference)
import functools

import jax
import jax.numpy as jnp
from jax import lax
from jax.experimental import pallas as pl
from jax.experimental.pallas import tpu as pltpu
from jax.experimental.pallas import tpu_sc as plsc

NC = 2
NS = 16
LN = 16
NW = NC * NS

CH = 80

_SC_PARAMS = pltpu.CompilerParams(
    needs_layout_passes=False, use_tc_tiling_on_sc=False)
_MESH = dict(core_axis_name="c", subcore_axis_name="s")


def _deg_kernel(E, NP):
    EW = E // NW

    @functools.partial(
        pl.kernel,
        out_type=jax.ShapeDtypeStruct((NW, NP), jnp.float32),
        mesh=plsc.VectorSubcoreMesh(**_MESH),
        compiler_params=_SC_PARAMS,
        scratch_types=[
            pltpu.VMEM((EW,), jnp.int32),
            pltpu.VMEM((EW,), jnp.float32),
            pltpu.VMEM((NP,), jnp.float32),
        ],
    )
    def k(dst_hbm, w_hbm, out_hbm, dst_v, w_v, deg_v):
        cid = lax.axis_index("c")
        sid = lax.axis_index("s")
        wid = sid * NC + cid

        def zero_body(i, _):
            deg_v[pl.ds(i * LN, LN)] = jnp.zeros((LN,), jnp.float32)
            return 0

        lax.fori_loop(0, NP // LN, zero_body, 0)

        pltpu.sync_copy(dst_hbm.at[pl.ds(wid * EW, EW)], dst_v)
        pltpu.sync_copy(w_hbm.at[pl.ds(wid * EW, EW)], w_v)

        def body(i, _):
            dv = dst_v[pl.ds(i * LN, LN)]
            wv = w_v[pl.ds(i * LN, LN)]
            plsc.addupdate_scatter(deg_v, [dv], wv)
            return 0

        lax.fori_loop(0, EW // LN, body, 0)
        pltpu.sync_copy(deg_v, out_hbm.at[wid])

    return k


def _norm_kernel(NP):
    R = NP // 128

    def body(parts_ref, norm_ref):
        deg = jnp.sum(parts_ref[...], axis=0)
        norm_ref[...] = lax.rsqrt(jnp.maximum(deg, 1.0))

    return pl.pallas_call(
        body,
        out_shape=jax.ShapeDtypeStruct((R, 128), jnp.float32),
        in_specs=[pl.BlockSpec((NW, R, 128), lambda: (0, 0, 0))],
        out_specs=pl.BlockSpec((R, 128), lambda: (0, 0)),
    )


def _layer_kernel(Nsrc, E, NP, D, with_fold):
    EW = E // NW
    NCH = EW // CH
    SCH = 25
    ROWS_PER_TILE = NP // NS
    FB = 80
    NB = ROWS_PER_TILE // FB

    out_type = [jax.ShapeDtypeStruct((NC, NP, D), jnp.float32)]
    if with_fold:
        out_type.append(jax.ShapeDtypeStruct((E // CH, CH), jnp.float32))

    @functools.partial(
        pl.kernel,
        out_type=tuple(out_type),
        mesh=plsc.VectorSubcoreMesh(**_MESH),
        compiler_params=_SC_PARAMS,
        scratch_types=[
            pltpu.VMEM((SCH, CH), jnp.int32),
            pltpu.VMEM((SCH, CH), jnp.int32),
            pltpu.VMEM((SCH, CH), jnp.float32),
            pltpu.VMEM((3, CH, D), jnp.float32),
            pltpu.VMEM((NP,), jnp.float32),
            pltpu.VMEM_SHARED((NP, D), jnp.float32),
            pltpu.SemaphoreType.DMA,
            pltpu.SemaphoreType.DMA,
            pltpu.SemaphoreType.DMA,
            pltpu.SemaphoreType.DMA,
            pltpu.SemaphoreType.DMA,
            pltpu.SemaphoreType.DMA,
        ],
    )
    def k(x_hbm, src_hbm, dst_hbm, s_hbm, norm_hbm, *rest):
        if with_fold:
            (out_hbm, sf_hbm, src_v, dst_v, s_v, rows_v, norm_v, acc_sh,
             gs0, gs1, gs2, ss0, ss1, ss2) = rest
        else:
            (out_hbm, src_v, dst_v, s_v, rows_v, norm_v, acc_sh,
             gs0, gs1, gs2, ss0, ss1, ss2) = rest
        cid = lax.axis_index("c")
        sid = lax.axis_index("s")
        wid = sid * NC + cid

        pltpu.sync_copy(norm_hbm, norm_v)

        def zrow(i, _):
            rows_v[0, i // (D // LN), pl.ds((i % (D // LN)) * LN, LN)] = (
                jnp.zeros((LN,), jnp.float32))
            return 0

        lax.fori_loop(0, CH * D // LN, zrow, 0)

        def zcp(b, _):
            base = sid * ROWS_PER_TILE + b * FB
            pltpu.sync_copy(rows_v.at[0, pl.ds(0, FB)],
                            acc_sh.at[pl.ds(base, FB)])
            return 0

        lax.fori_loop(0, NB, zcp, 0)
        plsc.subcore_barrier()

        gsem = (gs0, gs1, gs2)
        ssem = (ss0, ss1, ss2)

        def issue_gather(j, b):
            pltpu.async_copy(x_hbm.at[src_v.at[j]], rows_v.at[b], gsem[b])

        def wait_gather(b):
            pltpu.make_async_copy(
                x_hbm.at[pl.ds(0, CH)], rows_v.at[b], gsem[b]).wait()

        def issue_scatter(j, b):
            pltpu.async_copy(rows_v.at[b], acc_sh.at[dst_v.at[j]],
                             ssem[b], add=True)

        def wait_scatter(b):
            pltpu.make_async_copy(
                rows_v.at[b], acc_sh.at[pl.ds(0, CH)], ssem[b]).wait()

        def scale_chunk(j, b):
            def edge_body(g, _):
                wv = s_v[j, pl.ds(g * LN, LN)]
                for e in range(LN):
                    i = g * LN + e
                    ws = wv[e]
                    for t in range(D // LN):
                        rows_v[b, i, pl.ds(t * LN, LN)] = (
                            rows_v[b, i, pl.ds(t * LN, LN)] * ws)
                return 0

            lax.fori_loop(0, CH // LN, edge_body, 0)

        def sc_body(s, _):
            base_ch = wid * NCH + s * SCH
            pltpu.sync_copy(src_hbm.at[pl.ds(base_ch, SCH)], src_v)
            pltpu.sync_copy(dst_hbm.at[pl.ds(base_ch, SCH)], dst_v)
            pltpu.sync_copy(s_hbm.at[pl.ds(base_ch, SCH)], s_v)

            if with_fold:
                def fold_body(i, _):
                    j = i // (CH // LN)
                    t = i % (CH // LN)
                    sv = src_v[j, pl.ds(t * LN, LN)]
                    ns = plsc.load_gather(norm_v, [sv])
                    s_v[j, pl.ds(t * LN, LN)] = s_v[j, pl.ds(t * LN, LN)] * ns
                    return 0

                lax.fori_loop(0, SCH * CH // LN, fold_body, 0)
                pltpu.sync_copy(s_v, sf_hbm.at[pl.ds(base_ch, SCH)])

            issue_gather(0, 0)
            issue_gather(1, 1)

            def step(j, b, bprev):
                wait_gather(b)
                scale_chunk(j, b)
                issue_scatter(j, b)
                wait_scatter(bprev)
                issue_gather(jnp.minimum(j + 2, SCH - 1), bprev)

            def tri_body(q, _):
                j0 = 3 * q
                wait_gather(0)
                scale_chunk(j0, 0)
                issue_scatter(j0, 0)

                @pl.when(q > 0)
                def _():
                    wait_scatter(2)
                issue_gather(jnp.minimum(j0 + 2, SCH - 1), 2)
                step(j0 + 1, 1, 0)
                step(j0 + 2, 2, 1)
                return 0

            lax.fori_loop(0, (SCH - 1) // 3, tri_body, 0)

            wait_gather(0)
            scale_chunk(SCH - 1, 0)
            issue_scatter(SCH - 1, 0)
            wait_scatter(2)
            wait_gather(1)
            wait_scatter(0)
            return 0

        lax.fori_loop(0, NCH // SCH, sc_body, 0)
        plsc.subcore_barrier()

        def fin_blk(bb, _):
            base = sid * ROWS_PER_TILE + bb * FB
            pltpu.sync_copy(acc_sh.at[pl.ds(base, FB)],
                            rows_v.at[0, pl.ds(0, FB)])

            def fin_body(g, _):
                nv = norm_v[pl.ds(base + g * LN, LN)]
                for e in range(LN):
                    r = g * LN + e
                    ns = nv[e]
                    for t in range(D // LN):
                        rows_v[0, r, pl.ds(t * LN, LN)] = (
                            rows_v[0, r, pl.ds(t * LN, LN)] * ns)
                return 0

            lax.fori_loop(0, FB // LN, fin_body, 0)
            pltpu.sync_copy(rows_v.at[0, pl.ds(0, FB)],
                            out_hbm.at[cid, pl.ds(base, FB)])
            return 0

        lax.fori_loop(0, NB, fin_blk, 0)

    return k


def _add_kernel(NP, D):
    BR = 512

    def body(a_ref, b_ref, o_ref):
        o_ref[...] = a_ref[...] + b_ref[...]

    return pl.pallas_call(
        body,
        grid=(NP // BR,),
        out_shape=jax.ShapeDtypeStruct((NP, D), jnp.float32),
        in_specs=[pl.BlockSpec((BR, D), lambda i: (i, 0)),
                  pl.BlockSpec((BR, D), lambda i: (i, 0))],
        out_specs=pl.BlockSpec((BR, D), lambda i: (i, 0)),
    )


def _final_kernel(N, D):
    BR = 400

    def body(h_ref, h1_ref, p0_ref, p1_ref, o_ref):
        o_ref[...] = (h_ref[...] + h1_ref[...] + p0_ref[...] + p1_ref[...]) / 3.0

    return pl.pallas_call(
        body,
        grid=(N // BR,),
        out_shape=jax.ShapeDtypeStruct((N, D), jnp.float32),
        in_specs=[pl.BlockSpec((BR, D), lambda i: (i, 0)),
                  pl.BlockSpec((BR, D), lambda i: (i, 0)),
                  pl.BlockSpec((BR, D), lambda i: (i, 0)),
                  pl.BlockSpec((BR, D), lambda i: (i, 0))],
        out_specs=pl.BlockSpec((BR, D), lambda i: (i, 0)),
    )


def kernel(h, edge_index, edge_weight):
    N, D = h.shape
    E = edge_index.shape[1]
    NP = 10240 if N == 10000 else ((N + 16 * NW - 1) // (16 * NW)) * (16 * NW)

    src = edge_index[0].reshape(E // CH, CH)
    dst = edge_index[1].reshape(E // CH, CH)
    w2d = edge_weight.reshape(E // CH, CH)

    deg_parts = _deg_kernel(E, NP)(edge_index[1], edge_weight)
    norm2d = _norm_kernel(NP)(deg_parts.reshape(NW, NP // 128, 128))
    norm = norm2d.reshape(NP)

    p1, sfold2d = _layer_kernel(N, E, NP, D, True)(h, src, dst, w2d, norm)
    h1 = _add_kernel(NP, D)(p1[0], p1[1])
    (p2,) = _layer_kernel(NP, E, NP, D, False)(h1, src, dst, sfold2d, norm)

    return _final_kernel(N, D)(h, h1[:N], p2[0, :N], p2[1, :N])


# speedup vs baseline: 1.0219x; 1.0219x over previous
import functools

import jax
import jax.numpy as jnp
from jax import lax
from jax.experimental import pallas as pl
from jax.experimental.pallas import tpu as pltpu
from jax.experimental.pallas import tpu_sc as plsc

NC = 2
NS = 16
LN = 16
NW = NC * NS

CH = 80

_SC_PARAMS = pltpu.CompilerParams(
    needs_layout_passes=False, use_tc_tiling_on_sc=False)
_MESH = dict(core_axis_name="c", subcore_axis_name="s")


def _deg_kernel(E, NP):
    EW = E // NW

    @functools.partial(
        pl.kernel,
        out_type=jax.ShapeDtypeStruct((NW, NP), jnp.float32),
        mesh=plsc.VectorSubcoreMesh(**_MESH),
        compiler_params=_SC_PARAMS,
        scratch_types=[
            pltpu.VMEM((EW,), jnp.int32),
            pltpu.VMEM((EW,), jnp.float32),
            pltpu.VMEM((NP,), jnp.float32),
        ],
    )
    def k(dst_hbm, w_hbm, out_hbm, dst_v, w_v, deg_v):
        cid = lax.axis_index("c")
        sid = lax.axis_index("s")
        wid = sid * NC + cid

        def zero_body(i, _):
            deg_v[pl.ds(i * LN, LN)] = jnp.zeros((LN,), jnp.float32)
            return 0

        lax.fori_loop(0, NP // LN, zero_body, 0)

        pltpu.sync_copy(dst_hbm.at[pl.ds(wid * EW, EW)], dst_v)
        pltpu.sync_copy(w_hbm.at[pl.ds(wid * EW, EW)], w_v)

        def body(i, _):
            dv = dst_v[pl.ds(i * LN, LN)]
            wv = w_v[pl.ds(i * LN, LN)]
            plsc.addupdate_scatter(deg_v, [dv], wv)
            return 0

        lax.fori_loop(0, EW // LN, body, 0)
        pltpu.sync_copy(deg_v, out_hbm.at[wid])

    return k


def _norm_kernel(NP):
    R = NP // 128

    def body(parts_ref, norm_ref):
        deg = jnp.sum(parts_ref[...], axis=0)
        norm_ref[...] = lax.rsqrt(jnp.maximum(deg, 1.0))

    return pl.pallas_call(
        body,
        out_shape=jax.ShapeDtypeStruct((R, 128), jnp.float32),
        in_specs=[pl.BlockSpec((NW, R, 128), lambda: (0, 0, 0))],
        out_specs=pl.BlockSpec((R, 128), lambda: (0, 0)),
    )


def _layer_kernel(Nsrc, E, NP, D, with_fold):
    EW = E // NW
    NCH = EW // CH
    SCH = 25
    ROWS_PER_TILE = NP // NS
    FB = 80
    NB = ROWS_PER_TILE // FB

    out_type = [jax.ShapeDtypeStruct((NC, NP, D), jnp.float32)]
    if with_fold:
        out_type.append(jax.ShapeDtypeStruct((E // CH, CH), jnp.float32))

    @functools.partial(
        pl.kernel,
        out_type=tuple(out_type),
        mesh=plsc.VectorSubcoreMesh(**_MESH),
        compiler_params=_SC_PARAMS,
        scratch_types=[
            pltpu.VMEM((SCH, CH), jnp.int32),
            pltpu.VMEM((SCH, CH), jnp.int32),
            pltpu.VMEM((SCH, CH), jnp.float32),
            pltpu.VMEM((3, CH, D), jnp.float32),
            pltpu.VMEM((NP,), jnp.float32),
            pltpu.VMEM_SHARED((NP, D), jnp.float32),
            pltpu.SemaphoreType.DMA,
            pltpu.SemaphoreType.DMA,
            pltpu.SemaphoreType.DMA,
            pltpu.SemaphoreType.DMA,
            pltpu.SemaphoreType.DMA,
            pltpu.SemaphoreType.DMA,
        ],
    )
    def k(x_hbm, src_hbm, dst_hbm, s_hbm, norm_hbm, *rest):
        if with_fold:
            (out_hbm, sf_hbm, src_v, dst_v, s_v, rows_v, norm_v, acc_sh,
             gs0, gs1, gs2, ss0, ss1, ss2) = rest
        else:
            (out_hbm, src_v, dst_v, s_v, rows_v, norm_v, acc_sh,
             gs0, gs1, gs2, ss0, ss1, ss2) = rest
        cid = lax.axis_index("c")
        sid = lax.axis_index("s")
        wid = sid * NC + cid

        pltpu.sync_copy(norm_hbm, norm_v)

        def zrow(i, _):
            rows_v[0, i // (D // LN), pl.ds((i % (D // LN)) * LN, LN)] = (
                jnp.zeros((LN,), jnp.float32))
            return 0

        lax.fori_loop(0, CH * D // LN, zrow, 0)

        def zcp(b, _):
            base = sid * ROWS_PER_TILE + b * FB
            pltpu.sync_copy(rows_v.at[0, pl.ds(0, FB)],
                            acc_sh.at[pl.ds(base, FB)])
            return 0

        lax.fori_loop(0, NB, zcp, 0)
        plsc.subcore_barrier()

        gsem = (gs0, gs1, gs2)
        ssem = (ss0, ss1, ss2)

        def issue_gather(j, b):
            pltpu.async_copy(x_hbm.at[src_v.at[j]], rows_v.at[b], gsem[b])

        def wait_gather(b):
            pltpu.make_async_copy(
                x_hbm.at[pl.ds(0, CH)], rows_v.at[b], gsem[b]).wait()

        def issue_scatter(j, b):
            pltpu.async_copy(rows_v.at[b], acc_sh.at[dst_v.at[j]],
                             ssem[b], add=True)

        def wait_scatter(b):
            pltpu.make_async_copy(
                rows_v.at[b], acc_sh.at[pl.ds(0, CH)], ssem[b]).wait()

        def scale_chunk(j, b):
            def edge_body(g, _):
                wv = s_v[j, pl.ds(g * LN, LN)]
                for e in range(LN):
                    i = g * LN + e
                    ws = wv[e]
                    for t in range(D // LN):
                        rows_v[b, i, pl.ds(t * LN, LN)] = (
                            rows_v[b, i, pl.ds(t * LN, LN)] * ws)
                return 0

            lax.fori_loop(0, CH // LN, edge_body, 0)

        def sc_body(s, _):
            base_ch = wid * NCH + s * SCH
            pltpu.sync_copy(src_hbm.at[pl.ds(base_ch, SCH)], src_v)
            pltpu.sync_copy(dst_hbm.at[pl.ds(base_ch, SCH)], dst_v)
            pltpu.sync_copy(s_hbm.at[pl.ds(base_ch, SCH)], s_v)

            if with_fold:
                def fold_body(i, _):
                    j = i // (CH // LN)
                    t = i % (CH // LN)
                    sv = src_v[j, pl.ds(t * LN, LN)]
                    ns = plsc.load_gather(norm_v, [sv])
                    s_v[j, pl.ds(t * LN, LN)] = s_v[j, pl.ds(t * LN, LN)] * ns
                    return 0

                lax.fori_loop(0, SCH * CH // LN, fold_body, 0)
                pltpu.sync_copy(s_v, sf_hbm.at[pl.ds(base_ch, SCH)])

            issue_gather(0, 0)
            issue_gather(1, 1)

            def step(j, b, bprev):
                wait_gather(b)
                scale_chunk(j, b)
                issue_scatter(j, b)
                wait_scatter(bprev)
                issue_gather(jnp.minimum(j + 2, SCH - 1), bprev)

            def tri_body(q, _):
                j0 = 3 * q
                wait_gather(0)
                scale_chunk(j0, 0)
                issue_scatter(j0, 0)

                @pl.when(q > 0)
                def _():
                    wait_scatter(2)
                issue_gather(jnp.minimum(j0 + 2, SCH - 1), 2)
                step(j0 + 1, 1, 0)
                step(j0 + 2, 2, 1)
                return 0

            lax.fori_loop(0, (SCH - 1) // 3, tri_body, 0)

            wait_gather(0)
            scale_chunk(SCH - 1, 0)
            issue_scatter(SCH - 1, 0)
            wait_scatter(2)
            wait_gather(1)
            wait_scatter(0)
            return 0

        lax.fori_loop(0, NCH // SCH, sc_body, 0)
        plsc.subcore_barrier()

        def fin_blk(bb, _):
            base = sid * ROWS_PER_TILE + bb * FB
            pltpu.sync_copy(acc_sh.at[pl.ds(base, FB)],
                            rows_v.at[0, pl.ds(0, FB)])

            def fin_body(g, _):
                nv = norm_v[pl.ds(base + g * LN, LN)]
                for e in range(LN):
                    r = g * LN + e
                    ns = nv[e]
                    for t in range(D // LN):
                        rows_v[0, r, pl.ds(t * LN, LN)] = (
                            rows_v[0, r, pl.ds(t * LN, LN)] * ns)
                return 0

            lax.fori_loop(0, FB // LN, fin_body, 0)
            pltpu.sync_copy(rows_v.at[0, pl.ds(0, FB)],
                            out_hbm.at[cid, pl.ds(base, FB)])
            return 0

        lax.fori_loop(0, NB, fin_blk, 0)

    return k


def _add_kernel(NP, D):
    BR = 512

    def body(a_ref, b_ref, o_ref):
        o_ref[...] = a_ref[...] + b_ref[...]

    return pl.pallas_call(
        body,
        grid=(NP // BR,),
        out_shape=jax.ShapeDtypeStruct((NP, D), jnp.float32),
        in_specs=[pl.BlockSpec((BR, D), lambda i: (i, 0)),
                  pl.BlockSpec((BR, D), lambda i: (i, 0))],
        out_specs=pl.BlockSpec((BR, D), lambda i: (i, 0)),
    )


def _final_kernel(N, D):
    BR = 400

    def body(h_ref, h1_ref, p2_ref, o_ref):
        o_ref[...] = (h_ref[...] + h1_ref[...]
                      + p2_ref[0] + p2_ref[1]) / 3.0

    def make(h, h1, p2):
        NPAD = h1.shape[0]
        return pl.pallas_call(
            body,
            grid=(N // BR,),
            out_shape=jax.ShapeDtypeStruct((N, D), jnp.float32),
            in_specs=[pl.BlockSpec((BR, D), lambda i: (i, 0)),
                      pl.BlockSpec((BR, D), lambda i: (i, 0)),
                      pl.BlockSpec((2, BR, D), lambda i: (0, i, 0))],
            out_specs=pl.BlockSpec((BR, D), lambda i: (i, 0)),
        )(h, h1, p2)

    return make


def kernel(h, edge_index, edge_weight):
    N, D = h.shape
    E = edge_index.shape[1]
    NP = 10240 if N == 10000 else ((N + 16 * NW - 1) // (16 * NW)) * (16 * NW)

    src = edge_index[0].reshape(E // CH, CH)
    dst = edge_index[1].reshape(E // CH, CH)
    w2d = edge_weight.reshape(E // CH, CH)

    deg_parts = _deg_kernel(E, NP)(edge_index[1], edge_weight)
    norm2d = _norm_kernel(NP)(deg_parts.reshape(NW, NP // 128, 128))
    norm = norm2d.reshape(NP)

    p1, sfold2d = _layer_kernel(N, E, NP, D, True)(h, src, dst, w2d, norm)
    h1 = _add_kernel(NP, D)(p1[0], p1[1])
    (p2,) = _layer_kernel(NP, E, NP, D, False)(h1, src, dst, sfold2d, norm)

    return _final_kernel(N, D)(h, h1, p2)
